# baseline (speedup 1.0000x reference)
import functools

import jax
import jax.numpy as jnp
from jax import lax
from jax.experimental import pallas as pl
from jax.experimental.pallas import tpu as pltpu
from jax.experimental.pallas import tpu_sc as plsc

NC = 2
NS = 16
L = 16
NW = NC * NS

K = 64
SUPER = 8
ROW_BLK = 400


def _concat_proj_body(x_ref, w_ref, o_ref):
    x = x_ref[...]
    p = jnp.dot(x, w_ref[...], preferred_element_type=jnp.float32)
    o_ref[...] = jnp.concatenate([p, x], axis=1).astype(jnp.bfloat16)


def _rela_body(x_ref, wr_ref, wqr_ref, wqrb_ref, cr_ref, aq_ref):
    x = x_ref[...]
    pr = jnp.dot(x, wr_ref[...], preferred_element_type=jnp.float32)
    cr_ref[...] = jnp.concatenate([pr, x], axis=1).astype(jnp.bfloat16)
    aq_ref[...] = (jnp.dot(x, wqr_ref[...], preferred_element_type=jnp.float32)
                   + wqrb_ref[...]).astype(jnp.bfloat16)


def _final_body(acc_ref, wh_ref, o_ref):
    a = acc_ref[0] + acc_ref[1]
    o_ref[...] = jnp.dot(a, wh_ref[...], preferred_element_type=jnp.float32)


def _edge_kernel_body(n_row, rows_per_tile, chunks_per_worker, attn_dim, in_dim,
                      pk_h, qrel_h, zer_h, cs_h, cr_h, aq_h,
                      wa_h, wb_h, acc_out,
                      pk_v, qr0, qr1, obj0, obj1, sobj, qrel_v, wa_v, wb_v,
                      cs0, cs1, cr0, cr1, aq0, aq1, msg0, msg1, tp_v, acc_sh,
                      sg0, sg1, ss0, ss1):
    cid = lax.axis_index("c")
    sid = lax.axis_index("s")
    wid = sid * NC + cid

    pltpu.sync_copy(qrel_h, qrel_v)
    pltpu.sync_copy(wa_h, wa_v)
    pltpu.sync_copy(wb_h, wb_v)

    rows0 = sid * rows_per_tile
    pltpu.sync_copy(zer_h.at[pl.ds(rows0, rows_per_tile)],
                    acc_sh.at[pl.ds(rows0, rows_per_tile)])
    plsc.subcore_barrier()

    base_chunk = wid * chunks_per_worker

    def stage_idx(si, qr_ref, obj_ref):
        sv = jnp.full((L,), si, jnp.int32)
        c3 = jnp.full((L,), 3, jnp.int32)
        c2 = jnp.full((L,), 2, jnp.int32)
        for g in range(K // L):
            lane = g * L + lax.iota(jnp.int32, L)
            rv = plsc.load_gather(pk_v, [sv, c3, lane])
            qv = plsc.load_gather(qrel_v, [rv])
            plsc.store_scatter(qr_ref, [lane], qv)
            ov = plsc.load_gather(pk_v, [sv, c2, lane])
            plsc.store_scatter(obj_ref, [lane], ov)

    def issue_gathers(si, qr_ref, cs_b, cr_b, aq_b, sem):
        d1 = pltpu.async_copy(cs_h.at[pk_v.at[si, 0]], cs_b, sem)
        d2 = pltpu.async_copy(cr_h.at[pk_v.at[si, 1]], cr_b, sem)
        d3 = pltpu.async_copy(aq_h.at[qr_ref], aq_b, sem)
        return d1, d2, d3

    def wait_gathers(qr_ref, cs_b, cr_b, aq_b, sem):
        pltpu.make_async_copy(cs_h.at[pk_v.at[0, 0]], cs_b, sem).wait()
        pltpu.make_async_copy(cr_h.at[pk_v.at[0, 1]], cr_b, sem).wait()
        pltpu.make_async_copy(aq_h.at[qr_ref], aq_b, sem).wait()

    def unpack2(ref, ee, col):
        return plsc.unpack(ref[ee, pl.ds(col, 2 * L)],
                           format=plsc.PackFormat.INTERLEAVED,
                           preferred_element_type=jnp.float32)

    def compute_chunk(cs_b, cr_b, aq_b, msg_b):
        wv = [wa_v[pl.ds(i * L, L)] for i in range(attn_dim // L)]
        wbv = wb_v[...]
        for sb in range(K // L):
            e0 = sb * L
            for j in range(L):
                ee = e0 + j
                p = None
                for i in range(attn_dim // (2 * L)):
                    cse, cso = unpack2(cs_b, ee, 2 * L * i)
                    cre, cro = unpack2(cr_b, ee, 2 * L * i)
                    aqe, aqo = unpack2(aq_b, ee, 2 * L * i)
                    te = jnp.maximum(cse + cre + aqe, 0.0)
                    to = jnp.maximum(cso + cro + aqo, 0.0)
                    q = te * wv[2 * i] + to * wv[2 * i + 1]
                    p = q if p is None else p + q
                plsc.store_scatter(tp_v, [lax.iota(jnp.int32, L),
                                          jnp.full((L,), j, jnp.int32)], p)
            s = tp_v[0, :]
            for i in range(1, L):
                s = s + tp_v[i, :]
            s = s + wbv
            alpha16 = 1.0 / (1.0 + jnp.exp(-s))
            for j in range(L):
                ee = e0 + j
                ab = jnp.broadcast_to(alpha16[j], (L,))
                for i in range(in_dim // (2 * L)):
                    hse, hso = unpack2(cs_b, ee, attn_dim + 2 * L * i)
                    hre, hro = unpack2(cr_b, ee, attn_dim + 2 * L * i)
                    msg_b[ee, pl.ds(2 * L * i, L)] = hse * hre * ab
                    msg_b[ee, pl.ds(2 * L * i + L, L)] = hso * hro * ab

    def issue_scatter(obj_ref, srow, msg_b, sem):
        for g in range(K // L):
            lane = g * L + lax.iota(jnp.int32, L)
            ov = plsc.load_gather(obj_ref, [lane])
            plsc.store_scatter(sobj, [jnp.full((L,), srow, jnp.int32), lane], ov)
        return pltpu.async_copy(msg_b, acc_sh.at[sobj.at[srow]], sem, add=True)

    def drain_scatter(srow, msg_b, sem):
        pltpu.make_async_copy(msg_b, acc_sh.at[sobj.at[srow]], sem).wait()

    pltpu.sync_copy(pk_h.at[pl.ds(base_chunk, SUPER)], pk_v)
    stage_idx(0, qr0, obj0)
    issue_gathers(0, qr0, cs0, cr0, aq0, sg0)

    def body(cc, carry):
        a = 2 * cc
        stage_idx(a % SUPER + 1, qr1, obj1)
        issue_gathers(a % SUPER + 1, qr1, cs1, cr1, aq1, sg1)
        wait_gathers(qr0, cs0, cr0, aq0, sg0)

        @pl.when(cc > 0)
        def _():
            drain_scatter(0, msg0, ss0)
        compute_chunk(cs0, cr0, aq0, msg0)
        issue_scatter(obj0, 0, msg0, ss0)

        @pl.when(a + 2 < chunks_per_worker)
        def _():
            @pl.when((a + 2) % SUPER == 0)
            def _():
                pltpu.sync_copy(pk_h.at[pl.ds(base_chunk + a + 2, SUPER)], pk_v)
            stage_idx((a + 2) % SUPER, qr0, obj0)
            issue_gathers((a + 2) % SUPER, qr0, cs0, cr0, aq0, sg0)

        wait_gathers(qr1, cs1, cr1, aq1, sg1)

        @pl.when(cc > 0)
        def _():
            drain_scatter(1, msg1, ss1)
        compute_chunk(cs1, cr1, aq1, msg1)
        issue_scatter(obj1, 1, msg1, ss1)
        return carry

    lax.fori_loop(0, chunks_per_worker // 2, body, 0)
    drain_scatter(0, msg0, ss0)
    drain_scatter(1, msg1, ss1)

    plsc.subcore_barrier()
    pltpu.sync_copy(acc_sh.at[pl.ds(rows0, rows_per_tile)],
                    acc_out.at[cid, pl.ds(rows0, rows_per_tile)])


def kernel(q_sub, q_rel, r_idx, hidden, edges, n_node, rela_embed, Ws_w, Wr_w,
           Wqr_w, Wqr_b, walpha_w, walpha_b, Wh_w):
    n = hidden.shape[0]
    in_dim = hidden.shape[1]
    attn_dim = Ws_w.shape[1]
    n_rel_rows = rela_embed.shape[0]
    e = edges.shape[0]
    nq = q_rel.shape[0]

    sub = edges[:, 0].astype(jnp.int32)
    rel = edges[:, 1].astype(jnp.int32)
    obj = jnp.minimum(edges[:, 2], n_node - 1).astype(jnp.int32)
    ridx = r_idx.astype(jnp.int32)

    n_row = -(-(n + 1) // (NS * 8)) * (NS * 8)
    rows_per_tile = n_row // NS
    dump_row = n

    cpw = -(-e // (NW * K * SUPER)) * SUPER
    e_pad = NW * K * cpw
    pad = e_pad - e
    if pad:
        sub = jnp.concatenate([sub, jnp.zeros((pad,), jnp.int32)])
        rel = jnp.concatenate([rel, jnp.zeros((pad,), jnp.int32)])
        obj = jnp.concatenate([obj, jnp.full((pad,), dump_row, jnp.int32)])
        ridx = jnp.concatenate([ridx, jnp.zeros((pad,), jnp.int32)])
    n_chunk = e_pad // K
    pk = jnp.stack([sub.reshape(n_chunk, K), rel.reshape(n_chunk, K),
                    obj.reshape(n_chunk, K), ridx.reshape(n_chunk, K)], axis=1)

    r_pad = -(-n_rel_rows // ROW_BLK) * ROW_BLK
    rela_p = jnp.zeros((r_pad, in_dim), jnp.float32).at[:n_rel_rows].set(rela_embed)

    def _interleave_perm(nd):
        p = []
        for blk in range(0, nd, 2 * L):
            p.extend(range(blk, blk + 2 * L, 2))
            p.extend(range(blk + 1, blk + 2 * L, 2))
        return jnp.asarray(p, jnp.int32)

    wa = walpha_w.reshape(attn_dim).astype(jnp.float32)[_interleave_perm(attn_dim)]
    wh_p = Wh_w[_interleave_perm(in_dim), :]
    wb = jnp.broadcast_to(walpha_b.reshape(1), (L,)).astype(jnp.float32)
    wqrb = Wqr_b.reshape(1, attn_dim)

    cs = pl.pallas_call(
        _concat_proj_body,
        grid=(n // ROW_BLK,),
        in_specs=[
            pl.BlockSpec((ROW_BLK, in_dim), lambda i: (i, 0)),
            pl.BlockSpec((in_dim, attn_dim), lambda i: (0, 0)),
        ],
        out_specs=pl.BlockSpec((ROW_BLK, attn_dim + in_dim), lambda i: (i, 0)),
        out_shape=jax.ShapeDtypeStruct((n, attn_dim + in_dim), jnp.bfloat16),
    )(hidden, Ws_w)

    cr, aq = pl.pallas_call(
        _rela_body,
        grid=(r_pad // ROW_BLK,),
        in_specs=[
            pl.BlockSpec((ROW_BLK, in_dim), lambda i: (i, 0)),
            pl.BlockSpec((in_dim, attn_dim), lambda i: (0, 0)),
            pl.BlockSpec((in_dim, attn_dim), lambda i: (0, 0)),
            pl.BlockSpec((1, attn_dim), lambda i: (0, 0)),
        ],
        out_specs=[
            pl.BlockSpec((ROW_BLK, attn_dim + in_dim), lambda i: (i, 0)),
            pl.BlockSpec((ROW_BLK, attn_dim), lambda i: (i, 0)),
        ],
        out_shape=[
            jax.ShapeDtypeStruct((r_pad, attn_dim + in_dim), jnp.bfloat16),
            jax.ShapeDtypeStruct((r_pad, attn_dim), jnp.bfloat16),
        ],
    )(rela_p, Wr_w, Wqr_w, wqrb)

    zeros = jnp.zeros((n_row, in_dim), jnp.float32)
    mesh = plsc.VectorSubcoreMesh(core_axis_name="c", subcore_axis_name="s")
    body = functools.partial(_edge_kernel_body, n_row, rows_per_tile, cpw,
                             attn_dim, in_dim)
    acc = pl.kernel(
        body,
        out_type=jax.ShapeDtypeStruct((NC, n_row, in_dim), jnp.float32),
        mesh=mesh,
        compiler_params=pltpu.CompilerParams(
            needs_layout_passes=False, use_tc_tiling_on_sc=False),
        scratch_types=[
            pltpu.VMEM((SUPER, 4, K), jnp.int32),
            pltpu.VMEM((K,), jnp.int32),
            pltpu.VMEM((K,), jnp.int32),
            pltpu.VMEM((K,), jnp.int32),
            pltpu.VMEM((K,), jnp.int32),
            pltpu.VMEM((2, K), jnp.int32),
            pltpu.VMEM((nq,), jnp.int32),
            pltpu.VMEM((attn_dim,), jnp.float32),
            pltpu.VMEM((L,), jnp.float32),
            pltpu.VMEM((K, attn_dim + in_dim), jnp.bfloat16),
            pltpu.VMEM((K, attn_dim + in_dim), jnp.bfloat16),
            pltpu.VMEM((K, attn_dim + in_dim), jnp.bfloat16),
            pltpu.VMEM((K, attn_dim + in_dim), jnp.bfloat16),
            pltpu.VMEM((K, attn_dim), jnp.bfloat16),
            pltpu.VMEM((K, attn_dim), jnp.bfloat16),
            pltpu.VMEM((K, in_dim), jnp.float32),
            pltpu.VMEM((K, in_dim), jnp.float32),
            pltpu.VMEM((L, L), jnp.float32),
            pltpu.VMEM_SHARED((n_row, in_dim), jnp.float32),
            pltpu.SemaphoreType.DMA,
            pltpu.SemaphoreType.DMA,
            pltpu.SemaphoreType.DMA,
            pltpu.SemaphoreType.DMA,
        ],
    )(pk, q_rel.astype(jnp.int32), zeros, cs, cr, aq, wa, wb)

    out = pl.pallas_call(
        _final_body,
        grid=(n // ROW_BLK,),
        in_specs=[
            pl.BlockSpec((NC, ROW_BLK, in_dim), lambda i: (0, i, 0)),
            pl.BlockSpec((in_dim, Wh_w.shape[1]), lambda i: (0, 0)),
        ],
        out_specs=pl.BlockSpec((ROW_BLK, Wh_w.shape[1]), lambda i: (i, 0)),
        out_shape=jax.ShapeDtypeStruct((n, Wh_w.shape[1]), jnp.float32),
    )(acc, wh_p)
    return out


# speedup vs baseline: 1.0658x; 1.0658x over previous
import functools

import jax
import jax.numpy as jnp
from jax import lax
from jax.experimental import pallas as pl
from jax.experimental.pallas import tpu as pltpu
from jax.experimental.pallas import tpu_sc as plsc

NC = 2
NS = 16
L = 16
NW = NC * NS

K = 32
SUPER = 16
ROW_BLK = 400


def _concat_proj_body(x_ref, w_ref, o_ref):
    x = x_ref[...]
    p = jnp.dot(x, w_ref[...], preferred_element_type=jnp.float32)
    o_ref[...] = jnp.concatenate([p, x], axis=1).astype(jnp.bfloat16)


def _rela_body(x_ref, wr_ref, wqr_ref, wqrb_ref, cr_ref, aq_ref):
    x = x_ref[...]
    pr = jnp.dot(x, wr_ref[...], preferred_element_type=jnp.float32)
    cr_ref[...] = jnp.concatenate([pr, x], axis=1).astype(jnp.bfloat16)
    aq_ref[...] = (jnp.dot(x, wqr_ref[...], preferred_element_type=jnp.float32)
                   + wqrb_ref[...]).astype(jnp.bfloat16)


def _final_body(acc_ref, wh_ref, o_ref):
    a = acc_ref[0] + acc_ref[1]
    o_ref[...] = jnp.dot(a, wh_ref[...], preferred_element_type=jnp.float32)


def _edge_kernel_body(n_row, rows_per_tile, chunks_per_worker, attn_dim, in_dim,
                      pk_h, qrel_h, zer_h, cs_h, cr_h, aq_h,
                      wa_h, wb_h, acc_out,
                      pk_v, qr0, qr1, obj0, obj1, sobj, qrel_v, wa_v, wb_v,
                      cs0, cs1, cr0, cr1, aq0, aq1, msg0, msg1, tp_v, acc_sh,
                      sg0, sg1, ss0, ss1):
    cid = lax.axis_index("c")
    sid = lax.axis_index("s")
    wid = sid * NC + cid

    pltpu.sync_copy(qrel_h, qrel_v)
    pltpu.sync_copy(wa_h, wa_v)
    pltpu.sync_copy(wb_h, wb_v)

    rows0 = sid * rows_per_tile
    pltpu.sync_copy(zer_h.at[pl.ds(rows0, rows_per_tile)],
                    acc_sh.at[pl.ds(rows0, rows_per_tile)])
    plsc.subcore_barrier()

    base_chunk = wid * chunks_per_worker

    def stage_idx(si, qr_ref, obj_ref):
        sv = jnp.full((L,), si, jnp.int32)
        c3 = jnp.full((L,), 3, jnp.int32)
        c2 = jnp.full((L,), 2, jnp.int32)
        for g in range(K // L):
            lane = g * L + lax.iota(jnp.int32, L)
            rv = plsc.load_gather(pk_v, [sv, c3, lane])
            qv = plsc.load_gather(qrel_v, [rv])
            plsc.store_scatter(qr_ref, [lane], qv)
            ov = plsc.load_gather(pk_v, [sv, c2, lane])
            plsc.store_scatter(obj_ref, [lane], ov)

    def issue_gathers(si, qr_ref, cs_b, cr_b, aq_b, sem):
        d1 = pltpu.async_copy(cs_h.at[pk_v.at[si, 0]], cs_b, sem)
        d2 = pltpu.async_copy(cr_h.at[pk_v.at[si, 1]], cr_b, sem)
        d3 = pltpu.async_copy(aq_h.at[qr_ref], aq_b, sem)
        return d1, d2, d3

    def wait_gathers(qr_ref, cs_b, cr_b, aq_b, sem):
        pltpu.make_async_copy(cs_h.at[pk_v.at[0, 0]], cs_b, sem).wait()
        pltpu.make_async_copy(cr_h.at[pk_v.at[0, 1]], cr_b, sem).wait()
        pltpu.make_async_copy(aq_h.at[qr_ref], aq_b, sem).wait()

    def unpack2(ref, ee, col):
        return plsc.unpack(ref[ee, pl.ds(col, 2 * L)],
                           format=plsc.PackFormat.INTERLEAVED,
                           preferred_element_type=jnp.float32)

    def compute_chunk(cs_b, cr_b, aq_b, msg_b):
        wv = [wa_v[pl.ds(i * L, L)] for i in range(attn_dim // L)]
        wbv = wb_v[...]
        for sb in range(K // L):
            e0 = sb * L
            for j in range(L):
                ee = e0 + j
                p = None
                for i in range(attn_dim // (2 * L)):
                    cse, cso = unpack2(cs_b, ee, 2 * L * i)
                    cre, cro = unpack2(cr_b, ee, 2 * L * i)
                    aqe, aqo = unpack2(aq_b, ee, 2 * L * i)
                    te = jnp.maximum(cse + cre + aqe, 0.0)
                    to = jnp.maximum(cso + cro + aqo, 0.0)
                    q = te * wv[2 * i] + to * wv[2 * i + 1]
                    p = q if p is None else p + q
                plsc.store_scatter(tp_v, [lax.iota(jnp.int32, L),
                                          jnp.full((L,), j, jnp.int32)], p)
            s = tp_v[0, :]
            for i in range(1, L):
                s = s + tp_v[i, :]
            s = s + wbv
            alpha16 = 1.0 / (1.0 + jnp.exp(-s))
            for j in range(L):
                ee = e0 + j
                ab = jnp.broadcast_to(alpha16[j], (L,))
                for i in range(in_dim // (2 * L)):
                    hse, hso = unpack2(cs_b, ee, attn_dim + 2 * L * i)
                    hre, hro = unpack2(cr_b, ee, attn_dim + 2 * L * i)
                    msg_b[ee, pl.ds(2 * L * i, L)] = hse * hre * ab
                    msg_b[ee, pl.ds(2 * L * i + L, L)] = hso * hro * ab

    def issue_scatter(obj_ref, srow, msg_b, sem):
        for g in range(K // L):
            lane = g * L + lax.iota(jnp.int32, L)
            ov = plsc.load_gather(obj_ref, [lane])
            plsc.store_scatter(sobj, [jnp.full((L,), srow, jnp.int32), lane], ov)
        return pltpu.async_copy(msg_b, acc_sh.at[sobj.at[srow]], sem, add=True)

    def drain_scatter(srow, msg_b, sem):
        pltpu.make_async_copy(msg_b, acc_sh.at[sobj.at[srow]], sem).wait()

    pltpu.sync_copy(pk_h.at[pl.ds(base_chunk, SUPER)], pk_v)
    stage_idx(0, qr0, obj0)
    issue_gathers(0, qr0, cs0, cr0, aq0, sg0)

    def body(cc, carry):
        a = 2 * cc
        stage_idx(a % SUPER + 1, qr1, obj1)
        issue_gathers(a % SUPER + 1, qr1, cs1, cr1, aq1, sg1)
        wait_gathers(qr0, cs0, cr0, aq0, sg0)

        @pl.when(cc > 0)
        def _():
            drain_scatter(0, msg0, ss0)
        compute_chunk(cs0, cr0, aq0, msg0)
        issue_scatter(obj0, 0, msg0, ss0)

        @pl.when(a + 2 < chunks_per_worker)
        def _():
            @pl.when((a + 2) % SUPER == 0)
            def _():
                pltpu.sync_copy(pk_h.at[pl.ds(base_chunk + a + 2, SUPER)], pk_v)
            stage_idx((a + 2) % SUPER, qr0, obj0)
            issue_gathers((a + 2) % SUPER, qr0, cs0, cr0, aq0, sg0)

        wait_gathers(qr1, cs1, cr1, aq1, sg1)

        @pl.when(cc > 0)
        def _():
            drain_scatter(1, msg1, ss1)
        compute_chunk(cs1, cr1, aq1, msg1)
        issue_scatter(obj1, 1, msg1, ss1)
        return carry

    lax.fori_loop(0, chunks_per_worker // 2, body, 0)
    drain_scatter(0, msg0, ss0)
    drain_scatter(1, msg1, ss1)

    plsc.subcore_barrier()
    pltpu.sync_copy(acc_sh.at[pl.ds(rows0, rows_per_tile)],
                    acc_out.at[cid, pl.ds(rows0, rows_per_tile)])


def kernel(q_sub, q_rel, r_idx, hidden, edges, n_node, rela_embed, Ws_w, Wr_w,
           Wqr_w, Wqr_b, walpha_w, walpha_b, Wh_w):
    n = hidden.shape[0]
    in_dim = hidden.shape[1]
    attn_dim = Ws_w.shape[1]
    n_rel_rows = rela_embed.shape[0]
    e = edges.shape[0]
    nq = q_rel.shape[0]

    sub = edges[:, 0].astype(jnp.int32)
    rel = edges[:, 1].astype(jnp.int32)
    obj = jnp.minimum(edges[:, 2], n_node - 1).astype(jnp.int32)
    ridx = r_idx.astype(jnp.int32)

    n_row = -(-(n + 1) // (NS * 8)) * (NS * 8)
    rows_per_tile = n_row // NS
    dump_row = n

    cpw = -(-e // (NW * K * SUPER)) * SUPER
    e_pad = NW * K * cpw
    pad = e_pad - e
    if pad:
        sub = jnp.concatenate([sub, jnp.zeros((pad,), jnp.int32)])
        rel = jnp.concatenate([rel, jnp.zeros((pad,), jnp.int32)])
        obj = jnp.concatenate([obj, jnp.full((pad,), dump_row, jnp.int32)])
        ridx = jnp.concatenate([ridx, jnp.zeros((pad,), jnp.int32)])
    n_chunk = e_pad // K
    pk = jnp.stack([sub.reshape(n_chunk, K), rel.reshape(n_chunk, K),
                    obj.reshape(n_chunk, K), ridx.reshape(n_chunk, K)], axis=1)

    r_pad = -(-n_rel_rows // ROW_BLK) * ROW_BLK
    rela_p = jnp.zeros((r_pad, in_dim), jnp.float32).at[:n_rel_rows].set(rela_embed)

    def _interleave_perm(nd):
        p = []
        for blk in range(0, nd, 2 * L):
            p.extend(range(blk, blk + 2 * L, 2))
            p.extend(range(blk + 1, blk + 2 * L, 2))
        return jnp.asarray(p, jnp.int32)

    wa = walpha_w.reshape(attn_dim).astype(jnp.float32)[_interleave_perm(attn_dim)]
    wh_p = Wh_w[_interleave_perm(in_dim), :]
    wb = jnp.broadcast_to(walpha_b.reshape(1), (L,)).astype(jnp.float32)
    wqrb = Wqr_b.reshape(1, attn_dim)

    cs = pl.pallas_call(
        _concat_proj_body,
        grid=(n // ROW_BLK,),
        in_specs=[
            pl.BlockSpec((ROW_BLK, in_dim), lambda i: (i, 0)),
            pl.BlockSpec((in_dim, attn_dim), lambda i: (0, 0)),
        ],
        out_specs=pl.BlockSpec((ROW_BLK, attn_dim + in_dim), lambda i: (i, 0)),
        out_shape=jax.ShapeDtypeStruct((n, attn_dim + in_dim), jnp.bfloat16),
    )(hidden, Ws_w)

    cr, aq = pl.pallas_call(
        _rela_body,
        grid=(r_pad // ROW_BLK,),
        in_specs=[
            pl.BlockSpec((ROW_BLK, in_dim), lambda i: (i, 0)),
            pl.BlockSpec((in_dim, attn_dim), lambda i: (0, 0)),
            pl.BlockSpec((in_dim, attn_dim), lambda i: (0, 0)),
            pl.BlockSpec((1, attn_dim), lambda i: (0, 0)),
        ],
        out_specs=[
            pl.BlockSpec((ROW_BLK, attn_dim + in_dim), lambda i: (i, 0)),
            pl.BlockSpec((ROW_BLK, attn_dim), lambda i: (i, 0)),
        ],
        out_shape=[
            jax.ShapeDtypeStruct((r_pad, attn_dim + in_dim), jnp.bfloat16),
            jax.ShapeDtypeStruct((r_pad, attn_dim), jnp.bfloat16),
        ],
    )(rela_p, Wr_w, Wqr_w, wqrb)

    zeros = jnp.zeros((n_row, in_dim), jnp.float32)
    mesh = plsc.VectorSubcoreMesh(core_axis_name="c", subcore_axis_name="s")
    body = functools.partial(_edge_kernel_body, n_row, rows_per_tile, cpw,
                             attn_dim, in_dim)
    acc = pl.kernel(
        body,
        out_type=jax.ShapeDtypeStruct((NC, n_row, in_dim), jnp.float32),
        mesh=mesh,
        compiler_params=pltpu.CompilerParams(
            needs_layout_passes=False, use_tc_tiling_on_sc=False),
        scratch_types=[
            pltpu.VMEM((SUPER, 4, K), jnp.int32),
            pltpu.VMEM((K,), jnp.int32),
            pltpu.VMEM((K,), jnp.int32),
            pltpu.VMEM((K,), jnp.int32),
            pltpu.VMEM((K,), jnp.int32),
            pltpu.VMEM((2, K), jnp.int32),
            pltpu.VMEM((nq,), jnp.int32),
            pltpu.VMEM((attn_dim,), jnp.float32),
            pltpu.VMEM((L,), jnp.float32),
            pltpu.VMEM((K, attn_dim + in_dim), jnp.bfloat16),
            pltpu.VMEM((K, attn_dim + in_dim), jnp.bfloat16),
            pltpu.VMEM((K, attn_dim + in_dim), jnp.bfloat16),
            pltpu.VMEM((K, attn_dim + in_dim), jnp.bfloat16),
            pltpu.VMEM((K, attn_dim), jnp.bfloat16),
            pltpu.VMEM((K, attn_dim), jnp.bfloat16),
            pltpu.VMEM((K, in_dim), jnp.float32),
            pltpu.VMEM((K, in_dim), jnp.float32),
            pltpu.VMEM((L, L), jnp.float32),
            pltpu.VMEM_SHARED((n_row, in_dim), jnp.float32),
            pltpu.SemaphoreType.DMA,
            pltpu.SemaphoreType.DMA,
            pltpu.SemaphoreType.DMA,
            pltpu.SemaphoreType.DMA,
        ],
    )(pk, q_rel.astype(jnp.int32), zeros, cs, cr, aq, wa, wb)

    out = pl.pallas_call(
        _final_body,
        grid=(n // ROW_BLK,),
        in_specs=[
            pl.BlockSpec((NC, ROW_BLK, in_dim), lambda i: (0, i, 0)),
            pl.BlockSpec((in_dim, Wh_w.shape[1]), lambda i: (0, 0)),
        ],
        out_specs=pl.BlockSpec((ROW_BLK, Wh_w.shape[1]), lambda i: (i, 0)),
        out_shape=jax.ShapeDtypeStruct((n, Wh_w.shape[1]), jnp.float32),
    )(acc, wh_p)
    return out


# speedup vs baseline: 1.1001x; 1.0322x over previous
import functools

import jax
import jax.numpy as jnp
from jax import lax
from jax.experimental import pallas as pl
from jax.experimental.pallas import tpu as pltpu
from jax.experimental.pallas import tpu_sc as plsc

NC = 2
NS = 16
L = 16
NW = NC * NS

K = 32
SUPER = 16
ROW_BLK = 400


def _concat_proj_body(x_ref, w_ref, o_ref):
    x = x_ref[...]
    p = jnp.dot(x, w_ref[...], preferred_element_type=jnp.float32)
    o_ref[...] = jnp.concatenate([p, x], axis=1).astype(jnp.bfloat16)


def _rela_body(x_ref, wr_ref, wqr_ref, wqrb_ref, cr_ref, aq_ref):
    x = x_ref[...]
    pr = jnp.dot(x, wr_ref[...], preferred_element_type=jnp.float32)
    cr_ref[...] = jnp.concatenate([pr, x], axis=1).astype(jnp.bfloat16)
    aq_ref[...] = (jnp.dot(x, wqr_ref[...], preferred_element_type=jnp.float32)
                   + wqrb_ref[...]).astype(jnp.bfloat16)


def _final_body(acc_ref, wh_ref, o_ref):
    a = acc_ref[0] + acc_ref[1]
    o_ref[...] = jnp.dot(a, wh_ref[...], preferred_element_type=jnp.float32)


def _edge_kernel_body(n_row, rows_per_tile, chunks_per_worker, attn_dim, in_dim,
                      pk_h, qrel_h, zer_h, cs_h, cr_h, aq_h,
                      wa_h, wb_h, acc_out,
                      pk_v, qr0, qr1, obj0, obj1, sobj, qrel_v, wa_v, wb_v,
                      cs0, cs1, cr0, cr1, aq0, aq1, msg0, msg1, tp_v, acc_sh,
                      sg0, sg1, ss0, ss1):
    cid = lax.axis_index("c")
    sid = lax.axis_index("s")
    wid = sid * NC + cid

    pltpu.sync_copy(qrel_h, qrel_v)
    pltpu.sync_copy(wa_h, wa_v)
    pltpu.sync_copy(wb_h, wb_v)

    rows0 = sid * rows_per_tile
    pltpu.sync_copy(zer_h.at[pl.ds(rows0, rows_per_tile)],
                    acc_sh.at[pl.ds(rows0, rows_per_tile)])
    plsc.subcore_barrier()

    base_chunk = wid * chunks_per_worker

    def stage_idx(si, qr_ref, obj_ref):
        c3 = jnp.full((L,), 3, jnp.int32)
        c2 = jnp.full((L,), 2, jnp.int32)
        for g in range(K // L):
            lane = si * K + g * L + lax.iota(jnp.int32, L)
            rv = plsc.load_gather(pk_v, [c3, lane])
            qv = plsc.load_gather(qrel_v, [rv])
            plsc.store_scatter(qr_ref, [g * L + lax.iota(jnp.int32, L)], qv)
            ov = plsc.load_gather(pk_v, [c2, lane])
            plsc.store_scatter(obj_ref, [g * L + lax.iota(jnp.int32, L)], ov)

    def issue_gathers(si, qr_ref, cs_b, cr_b, aq_b, sem):
        d1 = pltpu.async_copy(cs_h.at[pk_v.at[0, pl.ds(si * K, K)]], cs_b, sem)
        d2 = pltpu.async_copy(cr_h.at[pk_v.at[1, pl.ds(si * K, K)]], cr_b, sem)
        d3 = pltpu.async_copy(aq_h.at[qr_ref], aq_b, sem)
        return d1, d2, d3

    def wait_gathers(qr_ref, cs_b, cr_b, aq_b, sem):
        pltpu.make_async_copy(cs_h.at[pk_v.at[0, pl.ds(0, K)]], cs_b, sem).wait()
        pltpu.make_async_copy(cr_h.at[pk_v.at[1, pl.ds(0, K)]], cr_b, sem).wait()
        pltpu.make_async_copy(aq_h.at[qr_ref], aq_b, sem).wait()

    def unpack2(ref, ee, col):
        return plsc.unpack(ref[ee, pl.ds(col, 2 * L)],
                           format=plsc.PackFormat.INTERLEAVED,
                           preferred_element_type=jnp.float32)

    def compute_chunk(cs_b, cr_b, aq_b, msg_b):
        wv = [wa_v[pl.ds(i * L, L)] for i in range(attn_dim // L)]
        wbv = wb_v[...]
        for sb in range(K // L):
            e0 = sb * L
            for j in range(L):
                ee = e0 + j
                p = None
                for i in range(attn_dim // (2 * L)):
                    cse, cso = unpack2(cs_b, ee, 2 * L * i)
                    cre, cro = unpack2(cr_b, ee, 2 * L * i)
                    aqe, aqo = unpack2(aq_b, ee, 2 * L * i)
                    te = jnp.maximum(cse + cre + aqe, 0.0)
                    to = jnp.maximum(cso + cro + aqo, 0.0)
                    q = te * wv[2 * i] + to * wv[2 * i + 1]
                    p = q if p is None else p + q
                plsc.store_scatter(tp_v, [lax.iota(jnp.int32, L),
                                          jnp.full((L,), j, jnp.int32)], p)
            s = tp_v[0, :]
            for i in range(1, L):
                s = s + tp_v[i, :]
            s = s + wbv
            alpha16 = 1.0 / (1.0 + jnp.exp(-s))
            for j in range(L):
                ee = e0 + j
                ab = jnp.broadcast_to(alpha16[j], (L,))
                for i in range(in_dim // (2 * L)):
                    hse, hso = unpack2(cs_b, ee, attn_dim + 2 * L * i)
                    hre, hro = unpack2(cr_b, ee, attn_dim + 2 * L * i)
                    msg_b[ee, pl.ds(2 * L * i, L)] = hse * hre * ab
                    msg_b[ee, pl.ds(2 * L * i + L, L)] = hso * hro * ab

    def issue_scatter(obj_ref, srow, msg_b, sem):
        for g in range(K // L):
            lane = g * L + lax.iota(jnp.int32, L)
            ov = plsc.load_gather(obj_ref, [lane])
            plsc.store_scatter(sobj, [jnp.full((L,), srow, jnp.int32), lane], ov)
        return pltpu.async_copy(msg_b, acc_sh.at[sobj.at[srow]], sem, add=True)

    def drain_scatter(srow, msg_b, sem):
        pltpu.make_async_copy(msg_b, acc_sh.at[sobj.at[srow]], sem).wait()

    pltpu.sync_copy(pk_h.at[:, pl.ds(base_chunk * K, SUPER * K)], pk_v)
    stage_idx(0, qr0, obj0)
    issue_gathers(0, qr0, cs0, cr0, aq0, sg0)

    def body(cc, carry):
        a = 2 * cc
        stage_idx(a % SUPER + 1, qr1, obj1)
        issue_gathers(a % SUPER + 1, qr1, cs1, cr1, aq1, sg1)
        wait_gathers(qr0, cs0, cr0, aq0, sg0)

        @pl.when(cc > 0)
        def _():
            drain_scatter(0, msg0, ss0)
        compute_chunk(cs0, cr0, aq0, msg0)
        issue_scatter(obj0, 0, msg0, ss0)

        @pl.when(a + 2 < chunks_per_worker)
        def _():
            @pl.when((a + 2) % SUPER == 0)
            def _():
                pltpu.sync_copy(
                    pk_h.at[:, pl.ds((base_chunk + a + 2) * K, SUPER * K)], pk_v)
            stage_idx((a + 2) % SUPER, qr0, obj0)
            issue_gathers((a + 2) % SUPER, qr0, cs0, cr0, aq0, sg0)

        wait_gathers(qr1, cs1, cr1, aq1, sg1)

        @pl.when(cc > 0)
        def _():
            drain_scatter(1, msg1, ss1)
        compute_chunk(cs1, cr1, aq1, msg1)
        issue_scatter(obj1, 1, msg1, ss1)
        return carry

    lax.fori_loop(0, chunks_per_worker // 2, body, 0)
    drain_scatter(0, msg0, ss0)
    drain_scatter(1, msg1, ss1)

    plsc.subcore_barrier()
    pltpu.sync_copy(acc_sh.at[pl.ds(rows0, rows_per_tile)],
                    acc_out.at[cid, pl.ds(rows0, rows_per_tile)])


def kernel(q_sub, q_rel, r_idx, hidden, edges, n_node, rela_embed, Ws_w, Wr_w,
           Wqr_w, Wqr_b, walpha_w, walpha_b, Wh_w):
    n = hidden.shape[0]
    in_dim = hidden.shape[1]
    attn_dim = Ws_w.shape[1]
    n_rel_rows = rela_embed.shape[0]
    e = edges.shape[0]
    nq = q_rel.shape[0]

    sub = edges[:, 0].astype(jnp.int32)
    rel = edges[:, 1].astype(jnp.int32)
    obj = jnp.minimum(edges[:, 2], n_node - 1).astype(jnp.int32)
    ridx = r_idx.astype(jnp.int32)

    n_row = -(-(n + 1) // (NS * 8)) * (NS * 8)
    rows_per_tile = n_row // NS
    dump_row = n

    cpw = -(-e // (NW * K * SUPER)) * SUPER
    e_pad = NW * K * cpw
    pad = e_pad - e
    if pad:
        sub = jnp.concatenate([sub, jnp.zeros((pad,), jnp.int32)])
        rel = jnp.concatenate([rel, jnp.zeros((pad,), jnp.int32)])
        obj = jnp.concatenate([obj, jnp.full((pad,), dump_row, jnp.int32)])
        ridx = jnp.concatenate([ridx, jnp.zeros((pad,), jnp.int32)])
    pk = jnp.concatenate([sub, rel, obj, ridx]).reshape(4, e_pad)

    def _interleave_perm(nd):
        p = []
        for blk in range(0, nd, 2 * L):
            p.extend(range(blk, blk + 2 * L, 2))
            p.extend(range(blk + 1, blk + 2 * L, 2))
        return jnp.asarray(p, jnp.int32)

    wa = walpha_w.reshape(attn_dim).astype(jnp.float32)[_interleave_perm(attn_dim)]
    wh_p = Wh_w[_interleave_perm(in_dim), :]
    wb = jnp.broadcast_to(walpha_b.reshape(1), (L,)).astype(jnp.float32)
    wqrb = Wqr_b.reshape(1, attn_dim)

    cs = pl.pallas_call(
        _concat_proj_body,
        grid=(n // ROW_BLK,),
        in_specs=[
            pl.BlockSpec((ROW_BLK, in_dim), lambda i: (i, 0)),
            pl.BlockSpec((in_dim, attn_dim), lambda i: (0, 0)),
        ],
        out_specs=pl.BlockSpec((ROW_BLK, attn_dim + in_dim), lambda i: (i, 0)),
        out_shape=jax.ShapeDtypeStruct((n, attn_dim + in_dim), jnp.bfloat16),
    )(hidden, Ws_w)

    cr, aq = pl.pallas_call(
        _rela_body,
        grid=(-(-n_rel_rows // ROW_BLK),),
        in_specs=[
            pl.BlockSpec((ROW_BLK, in_dim), lambda i: (i, 0)),
            pl.BlockSpec((in_dim, attn_dim), lambda i: (0, 0)),
            pl.BlockSpec((in_dim, attn_dim), lambda i: (0, 0)),
            pl.BlockSpec((1, attn_dim), lambda i: (0, 0)),
        ],
        out_specs=[
            pl.BlockSpec((ROW_BLK, attn_dim + in_dim), lambda i: (i, 0)),
            pl.BlockSpec((ROW_BLK, attn_dim), lambda i: (i, 0)),
        ],
        out_shape=[
            jax.ShapeDtypeStruct((n_rel_rows, attn_dim + in_dim), jnp.bfloat16),
            jax.ShapeDtypeStruct((n_rel_rows, attn_dim), jnp.bfloat16),
        ],
    )(rela_embed, Wr_w, Wqr_w, wqrb)

    zeros = jnp.zeros((n_row, in_dim), jnp.float32)
    mesh = plsc.VectorSubcoreMesh(core_axis_name="c", subcore_axis_name="s")
    body = functools.partial(_edge_kernel_body, n_row, rows_per_tile, cpw,
                             attn_dim, in_dim)
    acc = pl.kernel(
        body,
        out_type=jax.ShapeDtypeStruct((NC, n_row, in_dim), jnp.float32),
        mesh=mesh,
        compiler_params=pltpu.CompilerParams(
            needs_layout_passes=False, use_tc_tiling_on_sc=False),
        scratch_types=[
            pltpu.VMEM((4, SUPER * K), jnp.int32),
            pltpu.VMEM((K,), jnp.int32),
            pltpu.VMEM((K,), jnp.int32),
            pltpu.VMEM((K,), jnp.int32),
            pltpu.VMEM((K,), jnp.int32),
            pltpu.VMEM((2, K), jnp.int32),
            pltpu.VMEM((nq,), jnp.int32),
            pltpu.VMEM((attn_dim,), jnp.float32),
            pltpu.VMEM((L,), jnp.float32),
            pltpu.VMEM((K, attn_dim + in_dim), jnp.bfloat16),
            pltpu.VMEM((K, attn_dim + in_dim), jnp.bfloat16),
            pltpu.VMEM((K, attn_dim + in_dim), jnp.bfloat16),
            pltpu.VMEM((K, attn_dim + in_dim), jnp.bfloat16),
            pltpu.VMEM((K, attn_dim), jnp.bfloat16),
            pltpu.VMEM((K, attn_dim), jnp.bfloat16),
            pltpu.VMEM((K, in_dim), jnp.float32),
            pltpu.VMEM((K, in_dim), jnp.float32),
            pltpu.VMEM((L, L), jnp.float32),
            pltpu.VMEM_SHARED((n_row, in_dim), jnp.float32),
            pltpu.SemaphoreType.DMA,
            pltpu.SemaphoreType.DMA,
            pltpu.SemaphoreType.DMA,
            pltpu.SemaphoreType.DMA,
        ],
    )(pk, q_rel.astype(jnp.int32), zeros, cs, cr, aq, wa, wb)

    out = pl.pallas_call(
        _final_body,
        grid=(n // ROW_BLK,),
        in_specs=[
            pl.BlockSpec((NC, ROW_BLK, in_dim), lambda i: (0, i, 0)),
            pl.BlockSpec((in_dim, Wh_w.shape[1]), lambda i: (0, 0)),
        ],
        out_specs=pl.BlockSpec((ROW_BLK, Wh_w.shape[1]), lambda i: (i, 0)),
        out_shape=jax.ShapeDtypeStruct((n, Wh_w.shape[1]), jnp.float32),
    )(acc, wh_p)
    return out
